# baseline (speedup 1.0000x reference)
import functools

import jax
import jax.numpy as jnp
from jax import lax
from jax.experimental import pallas as pl
from jax.experimental.pallas import tpu as pltpu
from jax.experimental.pallas import tpu_sc as plsc

NC = 2
NS = 16
CH = 128
NGRAPH = 64


@functools.partial(jax.jit, static_argnums=(2, 3, 4))
def _sc_agg(h, edges, n_nodes, nblk, kb):
    dw = h.shape[1]
    mesh = plsc.VectorSubcoreMesh(core_axis_name="c", subcore_axis_name="s",
                                  num_cores=NC, num_subcores=NS)
    rpt = (-(-(n_nodes // 8) // NS)) * 8
    rpt_last = n_nodes - (NS - 1) * rpt

    @functools.partial(
        pl.kernel,
        out_type=[jax.ShapeDtypeStruct((n_nodes, dw), jnp.float32),
                  jax.ShapeDtypeStruct((n_nodes, dw), jnp.float32)],
        mesh=mesh,
        scratch_types=[
            pltpu.VMEM_SHARED((n_nodes + 8, dw), jnp.float32),
            pltpu.VMEM((kb, 2, CH), jnp.int32),
            pltpu.VMEM((kb, 2, CH), jnp.int32),
            pltpu.VMEM((CH, dw), jnp.float32),
            pltpu.VMEM((CH, dw), jnp.float32),
            pltpu.SemaphoreType.DMA,
            pltpu.SemaphoreType.DMA,
            pltpu.SemaphoreType.DMA,
            pltpu.SemaphoreType.DMA,
        ],
    )
    def agg(h_hbm, e_hbm, out0, out1, acc, idx_a, idx_b,
            rows_a, rows_b, sem_a, sem_b, sem_ia, sem_ib):
        c = lax.axis_index("c")
        s = lax.axis_index("s")
        wid = c * NS + s
        pltpu.async_copy(e_hbm.at[wid, 0], idx_a, sem_ia)
        pltpu.async_copy(e_hbm.at[wid, 1], idx_b, sem_ib)

        @pl.when(s < NS - 1)
        def _():
            pltpu.sync_copy(h_hbm.at[pl.ds(s * rpt, rpt)],
                            acc.at[pl.ds(s * rpt, rpt)])

        @pl.when(s == NS - 1)
        def _():
            pltpu.sync_copy(h_hbm.at[pl.ds((NS - 1) * rpt, rpt_last)],
                            acc.at[pl.ds((NS - 1) * rpt, rpt_last)])

        plsc.subcore_barrier()

        def do_block(bb, idx, semi):
            pltpu.make_async_copy(e_hbm.at[wid, bb], idx, semi).wait()

            pltpu.async_copy(h_hbm.at[idx.at[0, 0]], rows_a, sem_a)
            pltpu.async_copy(h_hbm.at[idx.at[1, 0]], rows_b, sem_b)

            @pl.loop(0, kb, step=2)
            def _(k):
                pltpu.make_async_copy(h_hbm.at[idx.at[k, 0]], rows_a,
                                      sem_a).wait()
                pltpu.sync_copy(rows_a, acc.at[idx.at[k, 1]], add=True)

                @pl.when(k + 2 < kb)
                def _():
                    pltpu.async_copy(h_hbm.at[idx.at[k + 2, 0]], rows_a, sem_a)

                pltpu.make_async_copy(h_hbm.at[idx.at[k + 1, 0]], rows_b,
                                      sem_b).wait()
                pltpu.sync_copy(rows_b, acc.at[idx.at[k + 1, 1]], add=True)

                @pl.when(k + 3 < kb)
                def _():
                    pltpu.async_copy(h_hbm.at[idx.at[k + 3, 0]], rows_b, sem_b)

            @pl.when(bb + 2 < nblk)
            def _():
                pltpu.async_copy(e_hbm.at[wid, bb + 2], idx, semi)

        @pl.loop(0, nblk, step=2)
        def _(bb):
            do_block(bb, idx_a, sem_ia)
            do_block(bb + 1, idx_b, sem_ib)

        plsc.subcore_barrier()

        @pl.when(jnp.logical_and(s == 0, c == 0))
        def _():
            pltpu.sync_copy(acc.at[pl.ds(0, n_nodes)], out0)

        @pl.when(jnp.logical_and(s == 0, c == 1))
        def _():
            pltpu.sync_copy(acc.at[pl.ds(0, n_nodes)], out1)

    return agg(h, edges)


def _mlp_body(a0_ref, a1_ref, hp_ref, w1_ref, b1_ref, w2_ref, b2_ref,
              g_ref, be_ref, hpre_ref, stats_ref, acc_ref, *, n_nodes):
    i = pl.program_id(0)
    hin = a0_ref[...] + a1_ref[...] - hp_ref[...]
    t = jnp.dot(hin, w1_ref[...], preferred_element_type=jnp.float32)
    t = jax.nn.gelu(t + b1_ref[...])
    hpre = jnp.dot(t, w2_ref[...], preferred_element_type=jnp.float32)
    hpre = hpre + b2_ref[...]
    hpre_ref[...] = hpre
    ps = jnp.sum(hpre, axis=0)
    pq = jnp.sum(hpre * hpre, axis=0)

    @pl.when(i == 0)
    def _():
        acc_ref[...] = jnp.zeros_like(acc_ref)

    acc_ref[0] += ps
    acc_ref[1] += pq

    @pl.when(i == pl.num_programs(0) - 1)
    def _():
        mu = acc_ref[0] / n_nodes
        var = acc_ref[1] / n_nodes - mu * mu
        scale = g_ref[0] * lax.rsqrt(var + 1e-5)
        stats_ref[0] = scale
        stats_ref[1] = be_ref[0] - mu * scale
        stats_ref[2:] = jnp.zeros_like(stats_ref[2:])


def _tc_mlp(a0, a1, hprev, w1, b1, w2, b2, g, be, br):
    n_nodes, din = hprev.shape
    k = w1.shape[1]
    grid = (n_nodes // br,)
    row = lambda i: (i, 0)
    fix = lambda i: (0, 0)
    return pl.pallas_call(
        functools.partial(_mlp_body, n_nodes=n_nodes),
        grid=grid,
        in_specs=[
            pl.BlockSpec((br, din), row),
            pl.BlockSpec((br, din), row),
            pl.BlockSpec((br, din), row),
            pl.BlockSpec((din, k), fix),
            pl.BlockSpec((1, k), fix),
            pl.BlockSpec((k, k), fix),
            pl.BlockSpec((1, k), fix),
            pl.BlockSpec((1, k), fix),
            pl.BlockSpec((1, k), fix),
        ],
        out_specs=[
            pl.BlockSpec((br, k), row),
            pl.BlockSpec((8, k), fix),
        ],
        out_shape=[
            jax.ShapeDtypeStruct((n_nodes, k), jnp.float32),
            jax.ShapeDtypeStruct((8, k), jnp.float32),
        ],
        scratch_shapes=[pltpu.VMEM((8, k), jnp.float32)],
    )(a0, a1, hprev, w1, b1, w2, b2, g, be)


def _bn_body(hpre_ref, stats_ref, batch_ref, h_ref, p_ref):
    i = pl.program_id(0)
    hb = hpre_ref[...] * stats_ref[0] + stats_ref[1]
    hb = jax.nn.gelu(hb)
    h_ref[...] = hb
    b = batch_ref[0, 0]
    oh = (b[:, None] == lax.broadcasted_iota(jnp.int32, (b.shape[0], NGRAPH), 1))
    oh = oh.astype(jnp.float32)
    pp = lax.dot_general(oh, hb, (((0,), (0,)), ((), ())),
                         preferred_element_type=jnp.float32)

    @pl.when(i == 0)
    def _():
        p_ref[...] = pp

    @pl.when(i > 0)
    def _():
        p_ref[...] += pp


def _tc_bn(hpre, stats, batch3, br):
    n_nodes, k = hpre.shape
    grid = (n_nodes // br,)
    return pl.pallas_call(
        _bn_body,
        grid=grid,
        in_specs=[
            pl.BlockSpec((br, k), lambda i: (i, 0)),
            pl.BlockSpec((8, k), lambda i: (0, 0)),
            pl.BlockSpec((1, 1, br), lambda i: (i, 0, 0)),
        ],
        out_specs=[
            pl.BlockSpec((br, k), lambda i: (i, 0)),
            pl.BlockSpec((NGRAPH, k), lambda i: (0, 0)),
        ],
        out_shape=[
            jax.ShapeDtypeStruct((n_nodes, k), jnp.float32),
            jax.ShapeDtypeStruct((NGRAPH, k), jnp.float32),
        ],
    )(hpre, stats, batch3)


def _readout_body(p1_ref, p2_ref, p3_ref, wl1_ref, bl1_ref, wl2_ref, bl2_ref,
                  out_ref):
    pc = jnp.concatenate([p1_ref[...], p2_ref[...], p3_ref[...]], axis=1)
    hh = jnp.dot(pc, wl1_ref[...], preferred_element_type=jnp.float32)
    hh = jnp.maximum(hh + bl1_ref[...], 0.0)
    out = jnp.dot(hh, wl2_ref[...], preferred_element_type=jnp.float32)
    out_ref[...] = out + bl2_ref[...]


def _tc_readout(p1, p2, p3, wl1, bl1, wl2, bl2):
    c = wl2.shape[1]
    return pl.pallas_call(
        _readout_body,
        out_shape=jax.ShapeDtypeStruct((NGRAPH, c), jnp.float32),
    )(p1, p2, p3, wl1, bl1, wl2, bl2)


def kernel(x, edge_index, batch, W11, b11, W12, b12, g1, be1,
           W21, b21, W22, b22, g2, be2,
           W31, b31, W32, b32, g3, be3,
           Wl1, bl1, Wl2, bl2):
    n, d = x.shape
    e = edge_index.shape[1]
    nw = NC * NS
    kb = 20
    blk_edges = kb * CH
    nblk = -(-e // (nw * blk_edges))
    if nblk % 2:
        nblk += 1
    nchunk = nblk * kb
    epad = nw * nchunk * CH
    src = edge_index[0]
    dst = edge_index[1]
    if epad > e:
        pad = epad - e
        src = jnp.concatenate([src, jnp.zeros((pad,), jnp.int32)])
        dst = jnp.concatenate([dst, jnp.full((pad,), n, jnp.int32)])
    edges = jnp.stack([src.reshape(nw, nblk, kb, CH),
                       dst.reshape(nw, nblk, kb, CH)], axis=3)

    br = 1000
    batch3 = batch.reshape(n // br, 1, br)
    r2 = lambda v: v.reshape(1, -1)

    a0, a1 = _sc_agg(x, edges, n, nblk, kb)
    hpre1, st1 = _tc_mlp(a0, a1, x, W11, r2(b11), W12, r2(b12),
                         r2(g1), r2(be1), br)
    h1, p1 = _tc_bn(hpre1, st1, batch3, br)

    a0, a1 = _sc_agg(h1, edges, n, nblk, kb)
    hpre2, st2 = _tc_mlp(a0, a1, h1, W21, r2(b21), W22, r2(b22),
                         r2(g2), r2(be2), br)
    h2, p2 = _tc_bn(hpre2, st2, batch3, br)

    a0, a1 = _sc_agg(h2, edges, n, nblk, kb)
    hpre3, st3 = _tc_mlp(a0, a1, h2, W31, r2(b31), W32, r2(b32),
                         r2(g3), r2(be3), br)
    h3, p3 = _tc_bn(hpre3, st3, batch3, br)

    return _tc_readout(p1, p2, p3, Wl1, r2(bl1), Wl2, r2(bl2))


# speedup vs baseline: 3.1388x; 3.1388x over previous
import functools

import jax
import jax.numpy as jnp
from jax import lax
from jax.experimental import pallas as pl
from jax.experimental.pallas import tpu as pltpu
from jax.experimental.pallas import tpu_sc as plsc

NC = 2
NS = 16
CH = 128
KB = 20
NGRAPH = 64


@functools.partial(jax.jit, static_argnums=(2, 3))
def _sc_agg(h, edges, n_nodes, nblk):
    dw = h.shape[1]
    mesh = plsc.VectorSubcoreMesh(core_axis_name="c", subcore_axis_name="s",
                                  num_cores=NC, num_subcores=NS)
    rpt = (-(-(n_nodes // 8) // NS)) * 8
    rpt_last = n_nodes - (NS - 1) * rpt

    @functools.partial(
        pl.kernel,
        out_type=[jax.ShapeDtypeStruct((n_nodes, dw), jnp.float32),
                  jax.ShapeDtypeStruct((n_nodes, dw), jnp.float32)],
        mesh=mesh,
        scratch_types=[
            pltpu.VMEM_SHARED((n_nodes + 8, dw), jnp.float32),
            pltpu.VMEM((KB, 2, CH), jnp.int32),
            pltpu.VMEM((KB, 2, CH), jnp.int32),
            pltpu.VMEM((CH, dw), jnp.float32),
            pltpu.VMEM((CH, dw), jnp.float32),
            pltpu.SemaphoreType.DMA,
            pltpu.SemaphoreType.DMA,
            pltpu.SemaphoreType.DMA,
            pltpu.SemaphoreType.DMA,
        ],
    )
    def agg(h_hbm, e_hbm, out0, out1, acc, idx_a, idx_b,
            rows_a, rows_b, sem_a, sem_b, sem_ia, sem_ib):
        c = lax.axis_index("c")
        s = lax.axis_index("s")
        wid = c * NS + s
        pltpu.async_copy(e_hbm.at[wid, 0], idx_a, sem_ia)
        pltpu.async_copy(e_hbm.at[wid, 1], idx_b, sem_ib)

        @pl.when(s < NS - 1)
        def _():
            sl = pl.ds(s * rpt, rpt)
            pltpu.sync_copy(h_hbm.at[sl], acc.at[sl])

        @pl.when(s == NS - 1)
        def _():
            sl = pl.ds((NS - 1) * rpt, rpt_last)
            pltpu.sync_copy(h_hbm.at[sl], acc.at[sl])

        plsc.subcore_barrier()

        def do_block(bb, idx, semi):
            pltpu.make_async_copy(e_hbm.at[wid, bb], idx, semi).wait()

            pltpu.async_copy(h_hbm.at[idx.at[0, 0]], rows_a, sem_a)
            pltpu.async_copy(h_hbm.at[idx.at[1, 0]], rows_b, sem_b)

            @pl.loop(0, KB, step=2)
            def _(k):
                pltpu.make_async_copy(h_hbm.at[idx.at[k, 0]], rows_a,
                                      sem_a).wait()
                pltpu.sync_copy(rows_a, acc.at[idx.at[k, 1]], add=True)

                @pl.when(k + 2 < KB)
                def _():
                    pltpu.async_copy(h_hbm.at[idx.at[k + 2, 0]], rows_a,
                                     sem_a)

                pltpu.make_async_copy(h_hbm.at[idx.at[k + 1, 0]], rows_b,
                                      sem_b).wait()
                pltpu.sync_copy(rows_b, acc.at[idx.at[k + 1, 1]], add=True)

                @pl.when(k + 3 < KB)
                def _():
                    pltpu.async_copy(h_hbm.at[idx.at[k + 3, 0]], rows_b,
                                     sem_b)

            @pl.when(bb + 2 < nblk)
            def _():
                pltpu.async_copy(e_hbm.at[wid, bb + 2], idx, semi)

        @pl.loop(0, nblk, step=2)
        def _(bb):
            do_block(bb, idx_a, sem_ia)
            do_block(bb + 1, idx_b, sem_ib)

        plsc.subcore_barrier()

        @pl.when(jnp.logical_and(s == 0, c == 0))
        def _():
            pltpu.sync_copy(acc.at[pl.ds(0, n_nodes)], out0)

        @pl.when(jnp.logical_and(s == 0, c == 1))
        def _():
            pltpu.sync_copy(acc.at[pl.ds(0, n_nodes)], out1)

    return agg(h, edges)


def _mlp_body(a0_ref, a1_ref, hp_ref, w1_ref, b1_ref, w2_ref, b2_ref,
              g_ref, be_ref, hpre_ref, stats_ref, acc_ref, *, n_nodes):
    i = pl.program_id(0)
    hin = a0_ref[...] + a1_ref[...] - hp_ref[...]
    t = jnp.dot(hin, w1_ref[...], preferred_element_type=jnp.float32)
    t = jax.nn.gelu(t + b1_ref[...])
    hpre = jnp.dot(t, w2_ref[...], preferred_element_type=jnp.float32)
    hpre = hpre + b2_ref[...]
    hpre_ref[...] = hpre
    ps = jnp.sum(hpre, axis=0)
    pq = jnp.sum(hpre * hpre, axis=0)

    @pl.when(i == 0)
    def _():
        acc_ref[...] = jnp.zeros_like(acc_ref)

    acc_ref[0] += ps
    acc_ref[1] += pq

    @pl.when(i == pl.num_programs(0) - 1)
    def _():
        mu = acc_ref[0] / n_nodes
        var = acc_ref[1] / n_nodes - mu * mu
        scale = g_ref[0] * lax.rsqrt(var + 1e-5)
        stats_ref[0] = scale
        stats_ref[1] = be_ref[0] - mu * scale
        stats_ref[2:] = jnp.zeros_like(stats_ref[2:])


def _tc_mlp(a0, a1, hprev, w1, b1, w2, b2, g, be, br):
    n_nodes, din = hprev.shape
    k = w1.shape[1]
    grid = (n_nodes // br,)
    row = lambda i: (i, 0)
    fix = lambda i: (0, 0)
    return pl.pallas_call(
        functools.partial(_mlp_body, n_nodes=n_nodes),
        grid=grid,
        in_specs=[
            pl.BlockSpec((br, din), row),
            pl.BlockSpec((br, din), row),
            pl.BlockSpec((br, din), row),
            pl.BlockSpec((din, k), fix),
            pl.BlockSpec((1, k), fix),
            pl.BlockSpec((k, k), fix),
            pl.BlockSpec((1, k), fix),
            pl.BlockSpec((1, k), fix),
            pl.BlockSpec((1, k), fix),
        ],
        out_specs=[
            pl.BlockSpec((br, k), row),
            pl.BlockSpec((8, k), fix),
        ],
        out_shape=[
            jax.ShapeDtypeStruct((n_nodes, k), jnp.float32),
            jax.ShapeDtypeStruct((8, k), jnp.float32),
        ],
        scratch_shapes=[pltpu.VMEM((8, k), jnp.float32)],
    )(a0, a1, hprev, w1, b1, w2, b2, g, be)


def _tc_bn(hpre, stats, batch3, br, want_h=True):
    n_nodes, k = hpre.shape
    grid = (n_nodes // br,)
    row = lambda i: (i, 0)
    fix = lambda i: (0, 0)
    if want_h:
        out_specs = [pl.BlockSpec((br, k), row),
                     pl.BlockSpec((NGRAPH, k), fix)]
        out_shape = [jax.ShapeDtypeStruct((n_nodes, k), jnp.float32),
                     jax.ShapeDtypeStruct((NGRAPH, k), jnp.float32)]
    else:
        out_specs = [pl.BlockSpec((NGRAPH, k), fix)]
        out_shape = [jax.ShapeDtypeStruct((NGRAPH, k), jnp.float32)]

    def body(hpre_ref, stats_ref, batch_ref, *outs):
        i = pl.program_id(0)
        hb = hpre_ref[...] * stats_ref[0] + stats_ref[1]
        hb = jax.nn.gelu(hb)
        if want_h:
            outs[0][...] = hb
        p_ref = outs[-1]
        b = batch_ref[0, 0]
        oh = (b[:, None] ==
              lax.broadcasted_iota(jnp.int32, (b.shape[0], NGRAPH), 1))
        oh = oh.astype(jnp.float32)
        pp = lax.dot_general(oh, hb, (((0,), (0,)), ((), ())),
                             preferred_element_type=jnp.float32)

        @pl.when(i == 0)
        def _():
            p_ref[...] = pp

        @pl.when(i > 0)
        def _():
            p_ref[...] += pp

    return pl.pallas_call(
        body,
        grid=grid,
        in_specs=[
            pl.BlockSpec((br, k), row),
            pl.BlockSpec((8, k), fix),
            pl.BlockSpec((1, 1, br), lambda i: (i, 0, 0)),
        ],
        out_specs=out_specs,
        out_shape=out_shape,
    )(hpre, stats, batch3)


def _readout_body(p1_ref, p2_ref, p3_ref, wl1_ref, bl1_ref, wl2_ref, bl2_ref,
                  out_ref):
    pc = jnp.concatenate([p1_ref[...], p2_ref[...], p3_ref[...]], axis=1)
    hh = jnp.dot(pc, wl1_ref[...], preferred_element_type=jnp.float32)
    hh = jnp.maximum(hh + bl1_ref[...], 0.0)
    out = jnp.dot(hh, wl2_ref[...], preferred_element_type=jnp.float32)
    out_ref[...] = out + bl2_ref[...]


def _tc_readout(p1, p2, p3, wl1, bl1, wl2, bl2):
    c = wl2.shape[1]
    return pl.pallas_call(
        _readout_body,
        out_shape=jax.ShapeDtypeStruct((NGRAPH, c), jnp.float32),
    )(p1, p2, p3, wl1, bl1, wl2, bl2)


def kernel(x, edge_index, batch, W11, b11, W12, b12, g1, be1,
           W21, b21, W22, b22, g2, be2,
           W31, b31, W32, b32, g3, be3,
           Wl1, bl1, Wl2, bl2):
    n, d = x.shape
    e = edge_index.shape[1]
    nw = NC * NS
    blk_edges = KB * CH
    nblk = -(-e // (nw * blk_edges))
    if nblk % 2:
        nblk += 1
    epad = nw * nblk * blk_edges
    src = edge_index[0]
    dst = edge_index[1]
    if epad > e:
        pad = epad - e
        pad_src = (jnp.arange(pad, dtype=jnp.int32) * 977) % n
        pad_dst = n + (jnp.arange(pad, dtype=jnp.int32) % 8)
        src = jnp.concatenate([src, pad_src])
        dst = jnp.concatenate([dst, pad_dst])
    edges = jnp.stack([src.reshape(nw, nblk, KB, CH),
                       dst.reshape(nw, nblk, KB, CH)], axis=3)

    br = 1000
    batch3 = batch.reshape(n // br, 1, br)
    r2 = lambda v: v.reshape(1, -1)

    a0, a1 = _sc_agg(x, edges, n, nblk)
    hpre1, st1 = _tc_mlp(a0, a1, x, W11, r2(b11), W12, r2(b12),
                         r2(g1), r2(be1), br)
    h1, p1 = _tc_bn(hpre1, st1, batch3, br)

    a0, a1 = _sc_agg(h1, edges, n, nblk)
    hpre2, st2 = _tc_mlp(a0, a1, h1, W21, r2(b21), W22, r2(b22),
                         r2(g2), r2(be2), br)
    h2, p2 = _tc_bn(hpre2, st2, batch3, br)

    a0, a1 = _sc_agg(h2, edges, n, nblk)
    hpre3, st3 = _tc_mlp(a0, a1, h2, W31, r2(b31), W32, r2(b32),
                         r2(g3), r2(be3), br)
    (p3,) = _tc_bn(hpre3, st3, batch3, br, want_h=False)

    return _tc_readout(p1, p2, p3, Wl1, r2(bl1), Wl2, r2(bl2))


# speedup vs baseline: 3.3035x; 1.0525x over previous
import functools

import jax
import jax.numpy as jnp
from jax import lax
from jax.experimental import pallas as pl
from jax.experimental.pallas import tpu as pltpu
from jax.experimental.pallas import tpu_sc as plsc

NC = 2
NS = 16
CH = 128
KB = 20
NGRAPH = 64


@functools.partial(jax.jit, static_argnums=(2, 3))
def _sc_agg(h, edges, n_nodes, nblk):
    dw = h.shape[1]
    mesh = plsc.VectorSubcoreMesh(core_axis_name="c", subcore_axis_name="s",
                                  num_cores=NC, num_subcores=NS)
    rpt = (-(-(n_nodes // 8) // NS)) * 8
    rpt_last = n_nodes - (NS - 1) * rpt

    @functools.partial(
        pl.kernel,
        out_type=[jax.ShapeDtypeStruct((n_nodes, dw), jnp.float32),
                  jax.ShapeDtypeStruct((n_nodes, dw), jnp.float32)],
        mesh=mesh,
        scratch_types=[
            pltpu.VMEM_SHARED((n_nodes + 8, dw), jnp.float32),
            pltpu.VMEM((KB, 2, CH), jnp.int32),
            pltpu.VMEM((KB, 2, CH), jnp.int32),
            pltpu.VMEM((CH, dw), jnp.float32),
            pltpu.VMEM((CH, dw), jnp.float32),
            pltpu.SemaphoreType.DMA,
            pltpu.SemaphoreType.DMA,
            pltpu.SemaphoreType.DMA,
            pltpu.SemaphoreType.DMA,
        ],
    )
    def agg(h_hbm, e_hbm, out0, out1, acc, idx_a, idx_b,
            rows_a, rows_b, sem_a, sem_b, sem_ia, sem_ib):
        c = lax.axis_index("c")
        s = lax.axis_index("s")
        wid = c * NS + s
        pltpu.async_copy(e_hbm.at[wid, 0], idx_a, sem_ia)
        pltpu.async_copy(e_hbm.at[wid, 1], idx_b, sem_ib)

        @pl.when(s < NS - 1)
        def _():
            sl = pl.ds(s * rpt, rpt)
            pltpu.sync_copy(h_hbm.at[sl], acc.at[sl])

        @pl.when(s == NS - 1)
        def _():
            sl = pl.ds((NS - 1) * rpt, rpt_last)
            pltpu.sync_copy(h_hbm.at[sl], acc.at[sl])

        plsc.subcore_barrier()

        def do_block(bb, idx, semi):
            pltpu.make_async_copy(e_hbm.at[wid, bb], idx, semi).wait()

            pltpu.async_copy(h_hbm.at[idx.at[0, 0]], rows_a, sem_a)
            pltpu.async_copy(h_hbm.at[idx.at[1, 0]], rows_b, sem_b)

            @pl.loop(0, KB, step=2)
            def _(k):
                pltpu.make_async_copy(h_hbm.at[idx.at[k, 0]], rows_a,
                                      sem_a).wait()
                pltpu.sync_copy(rows_a, acc.at[idx.at[k, 1]], add=True)

                @pl.when(k + 2 < KB)
                def _():
                    pltpu.async_copy(h_hbm.at[idx.at[k + 2, 0]], rows_a,
                                     sem_a)

                pltpu.make_async_copy(h_hbm.at[idx.at[k + 1, 0]], rows_b,
                                      sem_b).wait()
                pltpu.sync_copy(rows_b, acc.at[idx.at[k + 1, 1]], add=True)

                @pl.when(k + 3 < KB)
                def _():
                    pltpu.async_copy(h_hbm.at[idx.at[k + 3, 0]], rows_b,
                                     sem_b)

            @pl.when(bb + 2 < nblk)
            def _():
                pltpu.async_copy(e_hbm.at[wid, bb + 2], idx, semi)

        @pl.loop(0, nblk, step=2)
        def _(bb):
            do_block(bb, idx_a, sem_ia)
            do_block(bb + 1, idx_b, sem_ib)

        plsc.subcore_barrier()

        def wb(out):
            @pl.when(s < NS - 1)
            def _():
                sl = pl.ds(s * rpt, rpt)
                pltpu.sync_copy(acc.at[sl], out.at[sl])

            @pl.when(s == NS - 1)
            def _():
                sl = pl.ds((NS - 1) * rpt, rpt_last)
                pltpu.sync_copy(acc.at[sl], out.at[sl])

        @pl.when(c == 0)
        def _():
            wb(out0)

        @pl.when(c == 1)
        def _():
            wb(out1)

    return agg(h, edges)


def _tc_layer(a0, a1, hprev, w1, b1, w2, b2, g, be, batch2, br, want_h=True):
    n_nodes, din = hprev.shape
    k = w1.shape[1]
    nb = n_nodes // br
    grid = (nb,)
    row = lambda i: (i, 0)
    fix = lambda i: (0, 0)
    if want_h:
        out_specs = [pl.BlockSpec((n_nodes, k), fix),
                     pl.BlockSpec((NGRAPH, k), fix)]
        out_shape = [jax.ShapeDtypeStruct((n_nodes, k), jnp.float32),
                     jax.ShapeDtypeStruct((NGRAPH, k), jnp.float32)]
    else:
        out_specs = [pl.BlockSpec((NGRAPH, k), fix)]
        out_shape = [jax.ShapeDtypeStruct((NGRAPH, k), jnp.float32)]

    def body(a0_ref, a1_ref, hp_ref, w1_ref, b1_ref, w2_ref, b2_ref,
             g_ref, be_ref, batch_ref, *outs):
        hpre_ref = outs[-1]
        acc_ref = outs[-2]
        i = pl.program_id(0)
        hin = a0_ref[...] + a1_ref[...] - hp_ref[...]
        t = jnp.dot(hin, w1_ref[...], preferred_element_type=jnp.float32)
        t = jax.nn.gelu(t + b1_ref[...])
        hpre = jnp.dot(t, w2_ref[...], preferred_element_type=jnp.float32)
        hpre = hpre + b2_ref[...]
        hpre_ref[pl.ds(i * br, br), :] = hpre
        ps = jnp.sum(hpre, axis=0)
        pq = jnp.sum(hpre * hpre, axis=0)

        @pl.when(i == 0)
        def _():
            acc_ref[...] = jnp.zeros_like(acc_ref)

        acc_ref[0] += ps
        acc_ref[1] += pq

        @pl.when(i == nb - 1)
        def _():
            mu = acc_ref[0] / n_nodes
            var = acc_ref[1] / n_nodes - mu * mu
            scale = g_ref[0] * lax.rsqrt(var + 1e-5)
            shift = be_ref[0] - mu * scale
            hb = jax.nn.gelu(hpre_ref[...] * scale + shift)
            if want_h:
                outs[0][...] = hb
            p_ref = outs[0 if not want_h else 1]
            b = batch_ref[0]
            oh = (b[:, None] ==
                  lax.broadcasted_iota(jnp.int32, (n_nodes, NGRAPH), 1))
            p_ref[...] = lax.dot_general(oh.astype(jnp.float32), hb,
                                         (((0,), (0,)), ((), ())),
                                         preferred_element_type=jnp.float32)

    return pl.pallas_call(
        body,
        grid=grid,
        in_specs=[
            pl.BlockSpec((br, din), row),
            pl.BlockSpec((br, din), row),
            pl.BlockSpec((br, din), row),
            pl.BlockSpec((din, k), fix),
            pl.BlockSpec((1, k), fix),
            pl.BlockSpec((k, k), fix),
            pl.BlockSpec((1, k), fix),
            pl.BlockSpec((1, k), fix),
            pl.BlockSpec((1, k), fix),
            pl.BlockSpec((1, n_nodes), fix),
        ],
        out_specs=out_specs,
        out_shape=out_shape,
        scratch_shapes=[pltpu.VMEM((8, k), jnp.float32),
                        pltpu.VMEM((n_nodes, k), jnp.float32)],
    )(a0, a1, hprev, w1, b1, w2, b2, g, be, batch2)


def _readout_body(p1_ref, p2_ref, p3_ref, wl1_ref, bl1_ref, wl2_ref, bl2_ref,
                  out_ref):
    pc = jnp.concatenate([p1_ref[...], p2_ref[...], p3_ref[...]], axis=1)
    hh = jnp.dot(pc, wl1_ref[...], preferred_element_type=jnp.float32)
    hh = jnp.maximum(hh + bl1_ref[...], 0.0)
    out = jnp.dot(hh, wl2_ref[...], preferred_element_type=jnp.float32)
    out_ref[...] = out + bl2_ref[...]


def _tc_readout(p1, p2, p3, wl1, bl1, wl2, bl2):
    c = wl2.shape[1]
    return pl.pallas_call(
        _readout_body,
        out_shape=jax.ShapeDtypeStruct((NGRAPH, c), jnp.float32),
    )(p1, p2, p3, wl1, bl1, wl2, bl2)


def kernel(x, edge_index, batch, W11, b11, W12, b12, g1, be1,
           W21, b21, W22, b22, g2, be2,
           W31, b31, W32, b32, g3, be3,
           Wl1, bl1, Wl2, bl2):
    n, d = x.shape
    e = edge_index.shape[1]
    nw = NC * NS
    blk_edges = KB * CH
    nblk = -(-e // (nw * blk_edges))
    if nblk % 2:
        nblk += 1
    epad = nw * nblk * blk_edges
    src = edge_index[0]
    dst = edge_index[1]
    if epad > e:
        pad = epad - e
        pad_src = (jnp.arange(pad, dtype=jnp.int32) * 977) % n
        pad_dst = n + (jnp.arange(pad, dtype=jnp.int32) % 8)
        src = jnp.concatenate([src, pad_src])
        dst = jnp.concatenate([dst, pad_dst])
    edges = jnp.stack([src.reshape(nw, nblk, KB, CH),
                       dst.reshape(nw, nblk, KB, CH)], axis=3)

    br = 1000
    batch2 = batch.reshape(1, n)
    r2 = lambda v: v.reshape(1, -1)

    a0, a1 = _sc_agg(x, edges, n, nblk)
    h1, p1 = _tc_layer(a0, a1, x, W11, r2(b11), W12, r2(b12),
                       r2(g1), r2(be1), batch2, br)

    a0, a1 = _sc_agg(h1, edges, n, nblk)
    h2, p2 = _tc_layer(a0, a1, h1, W21, r2(b21), W22, r2(b22),
                       r2(g2), r2(be2), batch2, br)

    a0, a1 = _sc_agg(h2, edges, n, nblk)
    (p3,) = _tc_layer(a0, a1, h2, W31, r2(b31), W32, r2(b32),
                      r2(g3), r2(be3), batch2, br, want_h=False)

    return _tc_readout(p1, p2, p3, Wl1, r2(bl1), Wl2, r2(bl2))


# speedup vs baseline: 3.3519x; 1.0146x over previous
import functools

import jax
import jax.numpy as jnp
from jax import lax
from jax.experimental import pallas as pl
from jax.experimental.pallas import tpu as pltpu
from jax.experimental.pallas import tpu_sc as plsc

NC = 2
NS = 16
CH = 128
KB = 20
NGRAPH = 64


@functools.partial(jax.jit, static_argnums=(2, 3))
def _sc_agg(h, edges, n_nodes, nblk):
    dw = h.shape[1]
    mesh = plsc.VectorSubcoreMesh(core_axis_name="c", subcore_axis_name="s",
                                  num_cores=NC, num_subcores=NS)
    rpt = (-(-(n_nodes // 8) // NS)) * 8
    rpt_last = n_nodes - (NS - 1) * rpt

    @functools.partial(
        pl.kernel,
        out_type=[jax.ShapeDtypeStruct((n_nodes, dw), jnp.float32),
                  jax.ShapeDtypeStruct((n_nodes, dw), jnp.float32)],
        mesh=mesh,
        scratch_types=[
            pltpu.VMEM_SHARED((n_nodes + 8, dw), jnp.float32),
            pltpu.VMEM((KB, 2, CH), jnp.int32),
            pltpu.VMEM((KB, 2, CH), jnp.int32),
            pltpu.VMEM((CH, dw), jnp.float32),
            pltpu.VMEM((CH, dw), jnp.float32),
            pltpu.SemaphoreType.DMA,
            pltpu.SemaphoreType.DMA,
            pltpu.SemaphoreType.DMA,
            pltpu.SemaphoreType.DMA,
        ],
    )
    def agg(h_hbm, e_hbm, out0, out1, acc, idx_a, idx_b,
            rows_a, rows_b, sem_a, sem_b, sem_ia, sem_ib):
        c = lax.axis_index("c")
        s = lax.axis_index("s")
        wid = c * NS + s
        pltpu.async_copy(e_hbm.at[wid, 0], idx_a, sem_ia)
        pltpu.async_copy(e_hbm.at[wid, 1], idx_b, sem_ib)

        @pl.when(s < NS - 1)
        def _():
            sl = pl.ds(s * rpt, rpt)
            pltpu.sync_copy(h_hbm.at[sl], acc.at[sl])

        @pl.when(s == NS - 1)
        def _():
            sl = pl.ds((NS - 1) * rpt, rpt_last)
            pltpu.sync_copy(h_hbm.at[sl], acc.at[sl])

        def do_block(bb, idx, semi, first):
            pltpu.make_async_copy(e_hbm.at[wid, bb], idx, semi).wait()

            pltpu.async_copy(h_hbm.at[idx.at[0, 0]], rows_a, sem_a)
            pltpu.async_copy(h_hbm.at[idx.at[1, 0]], rows_b, sem_b)
            if first:
                plsc.subcore_barrier()

            @pl.loop(0, KB, step=2)
            def _(k):
                pltpu.make_async_copy(h_hbm.at[idx.at[k, 0]], rows_a,
                                      sem_a).wait()
                pltpu.sync_copy(rows_a, acc.at[idx.at[k, 1]], add=True)

                @pl.when(k + 2 < KB)
                def _():
                    pltpu.async_copy(h_hbm.at[idx.at[k + 2, 0]], rows_a,
                                     sem_a)

                pltpu.make_async_copy(h_hbm.at[idx.at[k + 1, 0]], rows_b,
                                      sem_b).wait()
                pltpu.sync_copy(rows_b, acc.at[idx.at[k + 1, 1]], add=True)

                @pl.when(k + 3 < KB)
                def _():
                    pltpu.async_copy(h_hbm.at[idx.at[k + 3, 0]], rows_b,
                                     sem_b)

            @pl.when(bb + 2 < nblk)
            def _():
                pltpu.async_copy(e_hbm.at[wid, bb + 2], idx, semi)

        do_block(0, idx_a, sem_ia, True)
        do_block(1, idx_b, sem_ib, False)

        @pl.loop(2, nblk, step=2)
        def _(bb):
            do_block(bb, idx_a, sem_ia, False)
            do_block(bb + 1, idx_b, sem_ib, False)

        plsc.subcore_barrier()

        def wb(out):
            @pl.when(s < NS - 1)
            def _():
                sl = pl.ds(s * rpt, rpt)
                pltpu.sync_copy(acc.at[sl], out.at[sl])

            @pl.when(s == NS - 1)
            def _():
                sl = pl.ds((NS - 1) * rpt, rpt_last)
                pltpu.sync_copy(acc.at[sl], out.at[sl])

        @pl.when(c == 0)
        def _():
            wb(out0)

        @pl.when(c == 1)
        def _():
            wb(out1)

    return agg(h, edges)


def _tc_layer(a0, a1, hprev, w1, b1, w2, b2, g, be, batch2, br,
              readout=None):
    n_nodes, din = hprev.shape
    k = w1.shape[1]
    nb = n_nodes // br
    grid = (nb,)
    row = lambda i: (i, 0)
    fix = lambda i: (0, 0)
    if readout is None:
        extra = ()
        extra_specs = []
        out_specs = [pl.BlockSpec((n_nodes, k), fix),
                     pl.BlockSpec((NGRAPH, k), fix)]
        out_shape = [jax.ShapeDtypeStruct((n_nodes, k), jnp.float32),
                     jax.ShapeDtypeStruct((NGRAPH, k), jnp.float32)]
    else:
        extra = tuple(readout)
        extra_specs = [pl.BlockSpec(p.shape, fix) for p in extra]
        nclass = extra[4].shape[1]
        out_specs = [pl.BlockSpec((NGRAPH, nclass), fix)]
        out_shape = [jax.ShapeDtypeStruct((NGRAPH, nclass), jnp.float32)]

    def body(a0_ref, a1_ref, hp_ref, w1_ref, b1_ref, w2_ref, b2_ref,
             g_ref, be_ref, batch_ref, *rest):
        hpre_ref = rest[-1]
        acc_ref = rest[-2]
        i = pl.program_id(0)
        hin = a0_ref[...] + a1_ref[...] - hp_ref[...]
        t = jnp.dot(hin, w1_ref[...], preferred_element_type=jnp.float32)
        t = jax.nn.gelu(t + b1_ref[...])
        hpre = jnp.dot(t, w2_ref[...], preferred_element_type=jnp.float32)
        hpre = hpre + b2_ref[...]
        hpre_ref[pl.ds(i * br, br), :] = hpre
        ps = jnp.sum(hpre, axis=0)
        pq = jnp.sum(hpre * hpre, axis=0)

        @pl.when(i == 0)
        def _():
            acc_ref[...] = jnp.zeros_like(acc_ref)

        acc_ref[0] += ps
        acc_ref[1] += pq

        @pl.when(i == nb - 1)
        def _():
            mu = acc_ref[0] / n_nodes
            var = acc_ref[1] / n_nodes - mu * mu
            scale = g_ref[0] * lax.rsqrt(var + 1e-5)
            shift = be_ref[0] - mu * scale
            hb = jax.nn.gelu(hpre_ref[...] * scale + shift)
            b = batch_ref[0]
            oh = (b[:, None] ==
                  lax.broadcasted_iota(jnp.int32, (n_nodes, NGRAPH), 1))
            pp = lax.dot_general(oh.astype(jnp.float32), hb,
                                 (((0,), (0,)), ((), ())),
                                 preferred_element_type=jnp.float32)
            if readout is None:
                h_ref, p_ref = rest[0], rest[1]
                h_ref[...] = hb
                p_ref[...] = pp
            else:
                p1_ref, p2_ref, wl1_ref, bl1_ref, wl2_ref, bl2_ref = rest[:6]
                out_ref = rest[6]
                pc = jnp.concatenate(
                    [p1_ref[...], p2_ref[...], pp], axis=1)
                hh = jnp.dot(pc, wl1_ref[...],
                             preferred_element_type=jnp.float32)
                hh = jnp.maximum(hh + bl1_ref[...], 0.0)
                out = jnp.dot(hh, wl2_ref[...],
                              preferred_element_type=jnp.float32)
                out_ref[...] = out + bl2_ref[...]

    return pl.pallas_call(
        body,
        grid=grid,
        in_specs=[
            pl.BlockSpec((br, din), row),
            pl.BlockSpec((br, din), row),
            pl.BlockSpec((br, din), row),
            pl.BlockSpec((din, k), fix),
            pl.BlockSpec((1, k), fix),
            pl.BlockSpec((k, k), fix),
            pl.BlockSpec((1, k), fix),
            pl.BlockSpec((1, k), fix),
            pl.BlockSpec((1, k), fix),
            pl.BlockSpec((1, n_nodes), fix),
        ] + extra_specs,
        out_specs=out_specs,
        out_shape=out_shape,
        scratch_shapes=[pltpu.VMEM((8, k), jnp.float32),
                        pltpu.VMEM((n_nodes, k), jnp.float32)],
    )(a0, a1, hprev, w1, b1, w2, b2, g, be, batch2, *extra)


def kernel(x, edge_index, batch, W11, b11, W12, b12, g1, be1,
           W21, b21, W22, b22, g2, be2,
           W31, b31, W32, b32, g3, be3,
           Wl1, bl1, Wl2, bl2):
    n, d = x.shape
    e = edge_index.shape[1]
    nw = NC * NS
    blk_edges = KB * CH
    nblk = -(-e // (nw * blk_edges))
    if nblk % 2:
        nblk += 1
    epad = nw * nblk * blk_edges
    src = edge_index[0]
    dst = edge_index[1]
    if epad > e:
        pad = epad - e
        pad_src = (jnp.arange(pad, dtype=jnp.int32) * 977) % n
        pad_dst = n + (jnp.arange(pad, dtype=jnp.int32) % 8)
        src = jnp.concatenate([src, pad_src])
        dst = jnp.concatenate([dst, pad_dst])
    edges = jnp.stack([src.reshape(nw, nblk, KB, CH),
                       dst.reshape(nw, nblk, KB, CH)], axis=3)

    br = 2000
    batch2 = batch.reshape(1, n)
    r2 = lambda v: v.reshape(1, -1)

    a0, a1 = _sc_agg(x, edges, n, nblk)
    h1, p1 = _tc_layer(a0, a1, x, W11, r2(b11), W12, r2(b12),
                       r2(g1), r2(be1), batch2, br)

    a0, a1 = _sc_agg(h1, edges, n, nblk)
    h2, p2 = _tc_layer(a0, a1, h1, W21, r2(b21), W22, r2(b22),
                       r2(g2), r2(be2), batch2, br)

    a0, a1 = _sc_agg(h2, edges, n, nblk)
    (out,) = _tc_layer(a0, a1, h2, W31, r2(b31), W32, r2(b32),
                       r2(g3), r2(be3), batch2, br,
                       readout=(p1, p2, Wl1, r2(bl1), Wl2, r2(bl2)))
    return out
